# Initial kernel scaffold; baseline (speedup 1.0000x reference)
import jax
import jax.numpy as jnp
from jax.experimental import pallas as pl


def kernel(X, edge_index, graph_sizes, W1, b1, W2, b2, Wf, bf):
    raise NotImplementedError("write your pallas kernel here")


# speedup vs baseline: 4.3175x; 4.3175x over previous
import functools

import jax
import jax.numpy as jnp
from jax import lax
from jax.experimental import pallas as pl
from jax.experimental.pallas import tpu as pltpu
from jax.experimental.pallas import tpu_sc as plsc

_NC = 2
_NS = 16
_NW = _NC * _NS
_CH = 128


def _make_sc_agg(N, d, CPT):
    mesh = plsc.VectorSubcoreMesh(core_axis_name="c", subcore_axis_name="s")
    stride = (N // _NS) // 8 * 8
    width = N - stride * (_NS - 1)
    assert width % 8 == 0 and width >= stride
    NP = N + 8

    @functools.partial(
        pl.kernel,
        out_type=jax.ShapeDtypeStruct((_NC * N, d), jnp.float32),
        mesh=mesh,
        scratch_types=[
            pltpu.VMEM((CPT, _CH), jnp.int32),
            pltpu.VMEM((CPT, _CH), jnp.int32),
            pltpu.VMEM((_CH, d), jnp.float32),
            pltpu.VMEM_SHARED((NP, d), jnp.float32),
            pltpu.SemaphoreType.DMA,
        ],
    )
    def agg(h_hbm, src_hbm, dst_hbm, out_hbm, src_v, dst_v, rows_v, agg_sh,
            sem):
        cid = lax.axis_index("c")
        sid = lax.axis_index("s")
        wid = sid * _NC + cid
        pltpu.sync_copy(src_hbm.at[wid], src_v)
        pltpu.sync_copy(dst_hbm.at[wid], dst_v)
        pltpu.sync_copy(h_hbm.at[pl.ds(sid * stride, width)],
                        agg_sh.at[pl.ds(sid * stride, width)])
        plsc.subcore_barrier()

        def body(j, carry):
            pltpu.async_copy(h_hbm.at[src_v.at[j]], rows_v, sem).wait()
            pltpu.sync_copy(rows_v, agg_sh.at[dst_v.at[j]], add=True)
            return carry

        lax.fori_loop(0, CPT, body, 0)
        plsc.subcore_barrier()
        pltpu.sync_copy(agg_sh.at[pl.ds(sid * stride, width)],
                        out_hbm.at[pl.ds(cid * N + sid * stride, width)])

    return agg


def _conv_body(p0_ref, p1_ref, h_ref, w_ref, b_ref, o_ref):
    agg = p0_ref[...] + p1_ref[...] - h_ref[...]
    y = jnp.dot(agg, w_ref[...], preferred_element_type=jnp.float32)
    o_ref[...] = jnp.maximum(y + b_ref[...], 0.0)


def _conv(p0, p1, h, W, b2d):
    N, d = h.shape
    BN = 2000
    grid = (N // BN,)
    return pl.pallas_call(
        _conv_body,
        grid=grid,
        in_specs=[
            pl.BlockSpec((BN, d), lambda i: (i, 0)),
            pl.BlockSpec((BN, d), lambda i: (i, 0)),
            pl.BlockSpec((BN, d), lambda i: (i, 0)),
            pl.BlockSpec((d, d), lambda i: (0, 0)),
            pl.BlockSpec((1, d), lambda i: (0, 0)),
        ],
        out_specs=pl.BlockSpec((BN, d), lambda i: (i, 0)),
        out_shape=jax.ShapeDtypeStruct((N, d), jnp.float32),
    )(p0, p1, h, W, b2d)


def _final_body(h_ref, wf_ref, bf_ref, gs_ref, o_ref):
    s = jnp.dot(h_ref[...], wf_ref[...], preferred_element_type=jnp.float32)
    s = jax.nn.sigmoid(s + bf_ref[...])
    G = o_ref.shape[0]
    ps = s.reshape(G, s.shape[0] // G, s.shape[1]).sum(axis=1)
    o_ref[...] = ps / gs_ref[...]


def _final(h, Wfp, bfp, gs, G):
    N, d = h.shape
    dp = Wfp.shape[1]
    return pl.pallas_call(
        _final_body,
        out_shape=jax.ShapeDtypeStruct((G, dp), jnp.float32),
    )(h, Wfp, bfp, gs)


def kernel(X, edge_index, graph_sizes, W1, b1, W2, b2, Wf, bf):
    N, d = X.shape
    E = edge_index.shape[1]
    G = graph_sizes.shape[0]
    T = Wf.shape[1]

    CPT = -(-E // (_NW * _CH))
    E_pad = _NW * CPT * _CH
    pad = E_pad - E
    src = jnp.concatenate(
        [edge_index[0], jnp.zeros((pad,), jnp.int32)]).reshape(_NW, CPT, _CH)
    dst = jnp.concatenate(
        [edge_index[1], jnp.full((pad,), N, jnp.int32)]).reshape(_NW, CPT, _CH)

    agg_fn = _make_sc_agg(N, d, CPT)
    b1r = b1.reshape(1, d)
    b2r = b2.reshape(1, d)
    dp = 128
    Wfp = jnp.zeros((d, dp), jnp.float32).at[:, :T].set(Wf)
    bfp = jnp.zeros((1, dp), jnp.float32).at[0, :T].set(bf)
    gs = graph_sizes.astype(jnp.float32).reshape(G, 1)

    h = X
    for W, br in ((W1, b1r), (W2, b2r)):
        p2 = agg_fn(h, src, dst)
        h = _conv(p2[:N], p2[N:], h, W, br)
    pooled = _final(h, Wfp, bfp, gs, G)
    return pooled[:, :T]
